# Initial kernel scaffold; baseline (speedup 1.0000x reference)
import jax
import jax.numpy as jnp
from jax.experimental import pallas as pl


def kernel(x, edge_index, W0, b0, W1, b1, W2, b2):
    raise NotImplementedError("write your pallas kernel here")


# speedup vs baseline: 5.1819x; 5.1819x over previous
import functools

import jax
import jax.numpy as jnp
from jax import lax
from jax.experimental import pallas as pl
from jax.experimental.pallas import tpu as pltpu
from jax.experimental.pallas import tpu_sc as plsc

_NC = 2
_NS = 16
_NW = _NC * _NS
_LANES = 16

_EDGE_BATCH = 128
_ROW_CHUNK = 80


def _zero_vmem_rows(buf, n_rows, row_words):
    nv = row_words // _LANES

    def body(i, c):
        for j in range(nv):
            buf[i, pl.ds(j * _LANES, _LANES)] = jnp.zeros((_LANES,), jnp.float32)
        return c

    lax.fori_loop(0, n_rows, body, 0)


def _fill_vmem_rows(buf, n_rows, row_words, value):
    nv = row_words // _LANES

    def body(i, c):
        for j in range(nv):
            buf[i, pl.ds(j * _LANES, _LANES)] = jnp.full((_LANES,), value, jnp.float32)
        return c

    lax.fori_loop(0, n_rows, body, 0)


def _chunk_loop(n_chunks, sid, fn):
    rounds = (n_chunks + _NS - 1) // _NS

    def body(t, c):
        ch = t * _NS + sid

        @pl.when(ch < n_chunks)
        def _():
            fn(ch)

        return c

    lax.fori_loop(0, rounds, body, 0)


_DEG_W = 128


def _make_deg_kernel(n_nodes, n_edges):
    n_batches = n_edges // _EDGE_BATCH
    rounds = (n_batches + _NW - 1) // _NW
    n_chunks = n_nodes // _ROW_CHUNK
    assert n_nodes % _ROW_CHUNK == 0

    mesh = plsc.VectorSubcoreMesh(core_axis_name="c", subcore_axis_name="s")

    @functools.partial(
        pl.kernel,
        mesh=mesh,
        out_type=jax.ShapeDtypeStruct((_NC, n_nodes, _DEG_W), jnp.float32),
        scratch_types=[
            pltpu.VMEM_SHARED((n_nodes, _DEG_W), jnp.float32),
            pltpu.VMEM((_ROW_CHUNK, _DEG_W), jnp.float32),
            pltpu.VMEM((_EDGE_BATCH, _DEG_W), jnp.float32),
            pltpu.VMEM((_EDGE_BATCH, _DEG_W), jnp.float32),
            pltpu.VMEM((_EDGE_BATCH,), jnp.int32),
            pltpu.VMEM((_EDGE_BATCH,), jnp.int32),
        ],
    )
    def deg_kernel(src_hbm, dst_hbm, out_hbm, acc, bounce, ones_s, ones_d,
                   idx_s, idx_d):
        cid = lax.axis_index("c")
        sid = lax.axis_index("s")
        wid = sid * _NC + cid

        _zero_vmem_rows(bounce, _ROW_CHUNK, _DEG_W)

        def zero_chunk(ch):
            r0 = pl.multiple_of(ch * _ROW_CHUNK, 8)
            pltpu.sync_copy(bounce, acc.at[pl.ds(r0, _ROW_CHUNK)])

        _chunk_loop(n_chunks, sid, zero_chunk)

        half = _DEG_W // 2
        nv = _DEG_W // _LANES

        def fill(i, c):
            for j in range(nv):
                v = 1.0 if j * _LANES < half else 0.0
                ones_s[i, pl.ds(j * _LANES, _LANES)] = jnp.full(
                    (_LANES,), v, jnp.float32)
                ones_d[i, pl.ds(j * _LANES, _LANES)] = jnp.full(
                    (_LANES,), 1.0 - v, jnp.float32)
            return c

        lax.fori_loop(0, _EDGE_BATCH, fill, 0)
        plsc.subcore_barrier()

        def body(t, c):
            b = t * _NW + wid

            @pl.when(b < n_batches)
            def _():
                base = pl.multiple_of(b * _EDGE_BATCH, _EDGE_BATCH)
                pltpu.sync_copy(src_hbm.at[pl.ds(base, _EDGE_BATCH)], idx_s)
                pltpu.sync_copy(ones_s, acc.at[idx_s], add=True)
                pltpu.sync_copy(dst_hbm.at[pl.ds(base, _EDGE_BATCH)], idx_d)
                pltpu.sync_copy(ones_d, acc.at[idx_d], add=True)

            return c

        lax.fori_loop(0, rounds, body, 0)
        plsc.subcore_barrier()

        def drain_chunk(ch):
            r0 = pl.multiple_of(ch * _ROW_CHUNK, 8)
            pltpu.sync_copy(acc.at[pl.ds(r0, _ROW_CHUNK)], bounce)
            pltpu.sync_copy(bounce, out_hbm.at[cid, pl.ds(r0, _ROW_CHUNK)])

        _chunk_loop(n_chunks, sid, drain_chunk)

    return deg_kernel


def _make_scatter_kernel(n_nodes, n_edges, d):
    n_batches = n_edges // _EDGE_BATCH
    rounds = (n_batches + _NW - 1) // _NW
    n_chunks = n_nodes // _ROW_CHUNK
    assert n_nodes % _ROW_CHUNK == 0

    mesh = plsc.VectorSubcoreMesh(core_axis_name="c", subcore_axis_name="s")

    @functools.partial(
        pl.kernel,
        mesh=mesh,
        out_type=jax.ShapeDtypeStruct((_NC, n_nodes, d), jnp.float32),
        scratch_types=[
            pltpu.VMEM_SHARED((n_nodes, d), jnp.float32),
            pltpu.VMEM((_ROW_CHUNK, d), jnp.float32),
            pltpu.VMEM((_EDGE_BATCH, d), jnp.float32),
            pltpu.VMEM((_EDGE_BATCH,), jnp.int32),
            pltpu.VMEM((_EDGE_BATCH,), jnp.int32),
            pltpu.SemaphoreType.DMA,
        ],
    )
    def scatter_kernel(h_hbm, src_hbm, dst_hbm, out_hbm, acc, bounce, rows_v,
                       idx_s, idx_d, sem):
        cid = lax.axis_index("c")
        sid = lax.axis_index("s")
        wid = sid * _NC + cid

        _zero_vmem_rows(bounce, _ROW_CHUNK, d)

        def zero_chunk(ch):
            r0 = pl.multiple_of(ch * _ROW_CHUNK, 8)
            pltpu.sync_copy(bounce, acc.at[pl.ds(r0, _ROW_CHUNK)])

        _chunk_loop(n_chunks, sid, zero_chunk)
        plsc.subcore_barrier()

        def body(t, c):
            b = t * _NW + wid

            @pl.when(b < n_batches)
            def _():
                base = pl.multiple_of(b * _EDGE_BATCH, _EDGE_BATCH)
                pltpu.sync_copy(src_hbm.at[pl.ds(base, _EDGE_BATCH)], idx_s)
                pltpu.async_copy(h_hbm.at[idx_s], rows_v, sem).wait()
                pltpu.sync_copy(dst_hbm.at[pl.ds(base, _EDGE_BATCH)], idx_d)
                pltpu.sync_copy(rows_v, acc.at[idx_d], add=True)

            return c

        lax.fori_loop(0, rounds, body, 0)
        plsc.subcore_barrier()

        def drain_chunk(ch):
            r0 = pl.multiple_of(ch * _ROW_CHUNK, 8)
            pltpu.sync_copy(acc.at[pl.ds(r0, _ROW_CHUNK)], bounce)
            pltpu.sync_copy(bounce, out_hbm.at[cid, pl.ds(r0, _ROW_CHUNK)])

        _chunk_loop(n_chunks, sid, drain_chunk)

    return scatter_kernel


_ROW_BLOCK = 1000


def _mm_first_body(x_ref, deg_ref, w_ref, o_ref):
    onorm = lax.rsqrt(jnp.maximum(deg_ref[...], 1.0))
    o_ref[...] = jnp.dot(x_ref[...] * onorm, w_ref[...],
                         preferred_element_type=jnp.float32,
                         precision=lax.Precision.HIGHEST)


def _mm_first(x, out_deg, w):
    n, din = x.shape
    dout = w.shape[1]
    grid = (n // _ROW_BLOCK,)
    return pl.pallas_call(
        _mm_first_body,
        grid=grid,
        in_specs=[
            pl.BlockSpec((_ROW_BLOCK, din), lambda i: (i, 0)),
            pl.BlockSpec((_ROW_BLOCK, 1), lambda i: (i, 0)),
            pl.BlockSpec((din, dout), lambda i: (0, 0)),
        ],
        out_specs=pl.BlockSpec((_ROW_BLOCK, dout), lambda i: (i, 0)),
        out_shape=jax.ShapeDtypeStruct((n, dout), jnp.float32),
    )(x, out_deg, w)


def _mm_mid_body(a0_ref, a1_ref, ideg_ref, odeg_ref, b_ref, w_ref, o_ref):
    inorm = lax.rsqrt(jnp.maximum(ideg_ref[...], 1.0))
    onorm = lax.rsqrt(jnp.maximum(odeg_ref[...], 1.0))
    rst = (a0_ref[...] + a1_ref[...]) * inorm + b_ref[...]
    rst = jnp.maximum(rst, 0.0)
    o_ref[...] = jnp.dot(rst * onorm, w_ref[...],
                         preferred_element_type=jnp.float32,
                         precision=lax.Precision.HIGHEST)


def _mm_mid(a0, a1, in_deg, out_deg, b, w):
    n, din = a0.shape
    dout = w.shape[1]
    grid = (n // _ROW_BLOCK,)
    return pl.pallas_call(
        _mm_mid_body,
        grid=grid,
        in_specs=[
            pl.BlockSpec((_ROW_BLOCK, din), lambda i: (i, 0)),
            pl.BlockSpec((_ROW_BLOCK, din), lambda i: (i, 0)),
            pl.BlockSpec((_ROW_BLOCK, 1), lambda i: (i, 0)),
            pl.BlockSpec((_ROW_BLOCK, 1), lambda i: (i, 0)),
            pl.BlockSpec((1, din), lambda i: (0, 0)),
            pl.BlockSpec((din, dout), lambda i: (0, 0)),
        ],
        out_specs=pl.BlockSpec((_ROW_BLOCK, dout), lambda i: (i, 0)),
        out_shape=jax.ShapeDtypeStruct((n, dout), jnp.float32),
    )(a0, a1, in_deg, out_deg, b, w)


def _mm_last_body(dout, a0_ref, a1_ref, ideg_ref, b_ref, o_ref):
    inorm = lax.rsqrt(jnp.maximum(ideg_ref[...], 1.0))
    agg = (a0_ref[...] + a1_ref[...])[:, :dout]
    o_ref[...] = agg * inorm + b_ref[...]


def _mm_last(a0, a1, in_deg, b):
    n, dpad = a0.shape
    dout = b.shape[1]
    grid = (n // _ROW_BLOCK,)
    return pl.pallas_call(
        functools.partial(_mm_last_body, dout),
        grid=grid,
        in_specs=[
            pl.BlockSpec((_ROW_BLOCK, dpad), lambda i: (i, 0)),
            pl.BlockSpec((_ROW_BLOCK, dpad), lambda i: (i, 0)),
            pl.BlockSpec((_ROW_BLOCK, 1), lambda i: (i, 0)),
            pl.BlockSpec((1, dout), lambda i: (0, 0)),
        ],
        out_specs=pl.BlockSpec((_ROW_BLOCK, dout), lambda i: (i, 0)),
        out_shape=jax.ShapeDtypeStruct((n, dout), jnp.float32),
    )(a0, a1, in_deg, b)


def kernel(x, edge_index, W0, b0, W1, b1, W2, b2):
    n, din = x.shape
    e = edge_index.shape[1]
    hid = W0.shape[1]
    ncls = W2.shape[1]

    src = edge_index[0]
    dst = edge_index[1]

    deg_parts = _make_deg_kernel(n, e)(src, dst)
    out_deg = (deg_parts[0, :, 0] + deg_parts[1, :, 0]).reshape(n, 1)
    in_deg = (deg_parts[0, :, 64] + deg_parts[1, :, 64]).reshape(n, 1)

    scat_hid = _make_scatter_kernel(n, e, hid)

    W2p = jnp.pad(W2, ((0, 0), (0, hid - ncls)))

    h0 = _mm_first(x, out_deg, W0)
    p0 = scat_hid(h0, src, dst)
    h1 = _mm_mid(p0[0], p0[1], in_deg, out_deg, b0.reshape(1, -1), W1)
    p1 = scat_hid(h1, src, dst)
    h2 = _mm_mid(p1[0], p1[1], in_deg, out_deg, b1.reshape(1, -1), W2p)
    p2 = scat_hid(h2, src, dst)
    out = _mm_last(p2[0], p2[1], in_deg, b2.reshape(1, -1))
    return out


# speedup vs baseline: 8.1651x; 1.5757x over previous
import functools

import jax
import jax.numpy as jnp
from jax import lax
from jax.experimental import pallas as pl
from jax.experimental.pallas import tpu as pltpu
from jax.experimental.pallas import tpu_sc as plsc

_NC = 2
_NS = 16
_NW = _NC * _NS
_LANES = 16

_EDGE_BATCH = 128
_ROW_CHUNK = 40


def _zero_vmem_rows(buf, n_rows, row_words):
    nv = row_words // _LANES

    def body(i, c):
        for j in range(nv):
            buf[i, pl.ds(j * _LANES, _LANES)] = jnp.zeros((_LANES,), jnp.float32)
        return c

    lax.fori_loop(0, n_rows, body, 0)


def _fill_vmem_rows(buf, n_rows, row_words, value):
    nv = row_words // _LANES

    def body(i, c):
        for j in range(nv):
            buf[i, pl.ds(j * _LANES, _LANES)] = jnp.full((_LANES,), value, jnp.float32)
        return c

    lax.fori_loop(0, n_rows, body, 0)


def _chunk_loop(n_chunks, sid, fn):
    rounds = (n_chunks + _NS - 1) // _NS

    def body(t, c):
        ch = t * _NS + sid

        @pl.when(ch < n_chunks)
        def _():
            fn(ch)

        return c

    lax.fori_loop(0, rounds, body, 0)


_DEG_W = 128


def _make_deg_kernel(n_nodes, n_edges):
    n_batches = n_edges // _EDGE_BATCH
    rounds = (n_batches + _NW - 1) // _NW
    n_chunks = n_nodes // _ROW_CHUNK
    assert n_nodes % _ROW_CHUNK == 0

    mesh = plsc.VectorSubcoreMesh(core_axis_name="c", subcore_axis_name="s")

    @functools.partial(
        pl.kernel,
        mesh=mesh,
        out_type=jax.ShapeDtypeStruct((_NC, n_nodes, _DEG_W), jnp.float32),
        scratch_types=[
            pltpu.VMEM_SHARED((n_nodes, _DEG_W), jnp.float32),
            pltpu.VMEM((_ROW_CHUNK, _DEG_W), jnp.float32),
            pltpu.VMEM((_EDGE_BATCH, _DEG_W), jnp.float32),
            pltpu.VMEM((_EDGE_BATCH, _DEG_W), jnp.float32),
            pltpu.VMEM((_EDGE_BATCH,), jnp.int32),
            pltpu.VMEM((_EDGE_BATCH,), jnp.int32),
        ],
    )
    def deg_kernel(src_hbm, dst_hbm, out_hbm, acc, bounce, ones_s, ones_d,
                   idx_s, idx_d):
        cid = lax.axis_index("c")
        sid = lax.axis_index("s")
        wid = sid * _NC + cid

        _zero_vmem_rows(bounce, _ROW_CHUNK, _DEG_W)

        def zero_chunk(ch):
            r0 = pl.multiple_of(ch * _ROW_CHUNK, 8)
            pltpu.sync_copy(bounce, acc.at[pl.ds(r0, _ROW_CHUNK)])

        _chunk_loop(n_chunks, sid, zero_chunk)

        half = _DEG_W // 2
        nv = _DEG_W // _LANES

        def fill(i, c):
            for j in range(nv):
                v = 1.0 if j * _LANES < half else 0.0
                ones_s[i, pl.ds(j * _LANES, _LANES)] = jnp.full(
                    (_LANES,), v, jnp.float32)
                ones_d[i, pl.ds(j * _LANES, _LANES)] = jnp.full(
                    (_LANES,), 1.0 - v, jnp.float32)
            return c

        lax.fori_loop(0, _EDGE_BATCH, fill, 0)
        plsc.subcore_barrier()

        def body(t, c):
            b = t * _NW + wid

            @pl.when(b < n_batches)
            def _():
                base = pl.multiple_of(b * _EDGE_BATCH, _EDGE_BATCH)
                pltpu.sync_copy(src_hbm.at[pl.ds(base, _EDGE_BATCH)], idx_s)
                pltpu.sync_copy(ones_s, acc.at[idx_s], add=True)
                pltpu.sync_copy(dst_hbm.at[pl.ds(base, _EDGE_BATCH)], idx_d)
                pltpu.sync_copy(ones_d, acc.at[idx_d], add=True)

            return c

        lax.fori_loop(0, rounds, body, 0)
        plsc.subcore_barrier()

        def drain_chunk(ch):
            r0 = pl.multiple_of(ch * _ROW_CHUNK, 8)
            pltpu.sync_copy(acc.at[pl.ds(r0, _ROW_CHUNK)], bounce)
            pltpu.sync_copy(bounce, out_hbm.at[cid, pl.ds(r0, _ROW_CHUNK)])

        _chunk_loop(n_chunks, sid, drain_chunk)

    return deg_kernel


def _make_scatter_kernel(n_nodes, n_edges, d):
    n_batches = n_edges // _EDGE_BATCH
    assert n_edges % _EDGE_BATCH == 0
    per_tile = -(-n_batches // _NW) // 8 * 8 + 8
    last_tile = n_batches - per_tile * (_NW - 1)
    assert 0 < last_tile <= per_tile

    def blocks_of(total):
        out = []
        while total >= 24:
            out.append(24)
            total -= 24
        while total > 0:
            out.append(min(8, total))
            total -= min(8, total)
        assert all(b % 2 == 0 for b in out)
        return out

    blocks_main = blocks_of(per_tile)
    blocks_last = blocks_of(last_tile)
    max_blk = max(blocks_main)

    n_chunks = n_nodes // _ROW_CHUNK
    assert n_nodes % _ROW_CHUNK == 0

    mesh = plsc.VectorSubcoreMesh(core_axis_name="c", subcore_axis_name="s")

    @functools.partial(
        pl.kernel,
        mesh=mesh,
        out_type=jax.ShapeDtypeStruct((_NC, n_nodes, d), jnp.float32),
        scratch_types=[
            pltpu.VMEM_SHARED((n_nodes, d), jnp.float32),
            pltpu.VMEM((_ROW_CHUNK, d), jnp.float32),
            pltpu.VMEM((_EDGE_BATCH, d), jnp.float32),
            pltpu.VMEM((_EDGE_BATCH, d), jnp.float32),
            pltpu.VMEM((max_blk, _EDGE_BATCH), jnp.int32),
            pltpu.VMEM((max_blk, _EDGE_BATCH), jnp.int32),
            pltpu.SemaphoreType.DMA,
            pltpu.SemaphoreType.DMA,
            pltpu.SemaphoreType.DMA,
            pltpu.SemaphoreType.DMA,
        ],
    )
    def scatter_kernel(h_hbm, src_hbm, dst_hbm, out_hbm, acc, bounce,
                       g0, g1, sblk, dblk, gs0, gs1, ss0, ss1):
        cid = lax.axis_index("c")
        sid = lax.axis_index("s")
        wid = sid * _NC + cid

        _zero_vmem_rows(bounce, _ROW_CHUNK, d)

        def zero_chunk(ch):
            r0 = pl.multiple_of(ch * _ROW_CHUNK, 8)
            pltpu.sync_copy(bounce, acc.at[pl.ds(r0, _ROW_CHUNK)])

        _chunk_loop(n_chunks, sid, zero_chunk)
        plsc.subcore_barrier()

        def run_block(row0, blk):
            pltpu.sync_copy(src_hbm.at[pl.ds(row0, blk)],
                            sblk.at[pl.ds(0, blk)])
            pltpu.sync_copy(dst_hbm.at[pl.ds(row0, blk)],
                            dblk.at[pl.ds(0, blk)])
            pltpu.async_copy(h_hbm.at[sblk.at[0]], g0, gs0)

            def pair(t, c):
                j0 = 2 * t
                pltpu.async_copy(h_hbm.at[sblk.at[j0 + 1]], g1, gs1)
                pltpu.make_async_copy(h_hbm.at[sblk.at[j0]], g0, gs0).wait()
                pltpu.async_copy(g0, acc.at[dblk.at[j0]], ss0, add=True)

                @pl.when(t < blk // 2 - 1)
                def _():
                    pltpu.make_async_copy(
                        g0, acc.at[dblk.at[j0]], ss0).wait()
                    pltpu.async_copy(h_hbm.at[sblk.at[j0 + 2]], g0, gs0)

                pltpu.make_async_copy(h_hbm.at[sblk.at[j0 + 1]], g1, gs1).wait()
                pltpu.async_copy(g1, acc.at[dblk.at[j0 + 1]], ss1, add=True)

                @pl.when(t < blk // 2 - 1)
                def _():
                    pltpu.make_async_copy(
                        g1, acc.at[dblk.at[j0 + 1]], ss1).wait()

                return c

            lax.fori_loop(0, blk // 2, pair, 0)
            pltpu.make_async_copy(g0, acc.at[dblk.at[0]], ss0).wait()
            pltpu.make_async_copy(g1, acc.at[dblk.at[0]], ss1).wait()

        @pl.when(wid < _NW - 1)
        def _():
            base = pl.multiple_of(wid * per_tile, 8)
            off = 0
            for blk in blocks_main:
                run_block(base + off, blk)
                off += blk

        @pl.when(wid == _NW - 1)
        def _():
            off = per_tile * (_NW - 1)
            for blk in blocks_last:
                run_block(off, blk)
                off += blk

        plsc.subcore_barrier()

        def drain_chunk(ch):
            r0 = pl.multiple_of(ch * _ROW_CHUNK, 8)
            pltpu.sync_copy(acc.at[pl.ds(r0, _ROW_CHUNK)], bounce)
            pltpu.sync_copy(bounce, out_hbm.at[cid, pl.ds(r0, _ROW_CHUNK)])

        _chunk_loop(n_chunks, sid, drain_chunk)

    return scatter_kernel


_ROW_BLOCK = 1000


def _mm_first_body(x_ref, deg_ref, w_ref, o_ref):
    onorm = lax.rsqrt(jnp.maximum(deg_ref[...], 1.0))
    o_ref[...] = jnp.dot(x_ref[...] * onorm, w_ref[...],
                         preferred_element_type=jnp.float32,
                         precision=lax.Precision.HIGHEST)


def _mm_first(x, out_deg, w):
    n, din = x.shape
    dout = w.shape[1]
    grid = (n // _ROW_BLOCK,)
    return pl.pallas_call(
        _mm_first_body,
        grid=grid,
        in_specs=[
            pl.BlockSpec((_ROW_BLOCK, din), lambda i: (i, 0)),
            pl.BlockSpec((_ROW_BLOCK, 1), lambda i: (i, 0)),
            pl.BlockSpec((din, dout), lambda i: (0, 0)),
        ],
        out_specs=pl.BlockSpec((_ROW_BLOCK, dout), lambda i: (i, 0)),
        out_shape=jax.ShapeDtypeStruct((n, dout), jnp.float32),
    )(x, out_deg, w)


def _mm_mid_body(a0_ref, a1_ref, ideg_ref, odeg_ref, b_ref, w_ref, o_ref):
    inorm = lax.rsqrt(jnp.maximum(ideg_ref[...], 1.0))
    onorm = lax.rsqrt(jnp.maximum(odeg_ref[...], 1.0))
    rst = (a0_ref[...] + a1_ref[...]) * inorm + b_ref[...]
    rst = jnp.maximum(rst, 0.0)
    o_ref[...] = jnp.dot(rst * onorm, w_ref[...],
                         preferred_element_type=jnp.float32,
                         precision=lax.Precision.HIGHEST)


def _mm_mid(a0, a1, in_deg, out_deg, b, w):
    n, din = a0.shape
    dout = w.shape[1]
    grid = (n // _ROW_BLOCK,)
    return pl.pallas_call(
        _mm_mid_body,
        grid=grid,
        in_specs=[
            pl.BlockSpec((_ROW_BLOCK, din), lambda i: (i, 0)),
            pl.BlockSpec((_ROW_BLOCK, din), lambda i: (i, 0)),
            pl.BlockSpec((_ROW_BLOCK, 1), lambda i: (i, 0)),
            pl.BlockSpec((_ROW_BLOCK, 1), lambda i: (i, 0)),
            pl.BlockSpec((1, din), lambda i: (0, 0)),
            pl.BlockSpec((din, dout), lambda i: (0, 0)),
        ],
        out_specs=pl.BlockSpec((_ROW_BLOCK, dout), lambda i: (i, 0)),
        out_shape=jax.ShapeDtypeStruct((n, dout), jnp.float32),
    )(a0, a1, in_deg, out_deg, b, w)


def _mm_last_body(dout, a0_ref, a1_ref, ideg_ref, b_ref, o_ref):
    inorm = lax.rsqrt(jnp.maximum(ideg_ref[...], 1.0))
    agg = (a0_ref[...] + a1_ref[...])[:, :dout]
    o_ref[...] = agg * inorm + b_ref[...]


def _mm_last(a0, a1, in_deg, b):
    n, dpad = a0.shape
    dout = b.shape[1]
    grid = (n // _ROW_BLOCK,)
    return pl.pallas_call(
        functools.partial(_mm_last_body, dout),
        grid=grid,
        in_specs=[
            pl.BlockSpec((_ROW_BLOCK, dpad), lambda i: (i, 0)),
            pl.BlockSpec((_ROW_BLOCK, dpad), lambda i: (i, 0)),
            pl.BlockSpec((_ROW_BLOCK, 1), lambda i: (i, 0)),
            pl.BlockSpec((1, dout), lambda i: (0, 0)),
        ],
        out_specs=pl.BlockSpec((_ROW_BLOCK, dout), lambda i: (i, 0)),
        out_shape=jax.ShapeDtypeStruct((n, dout), jnp.float32),
    )(a0, a1, in_deg, b)


def kernel(x, edge_index, W0, b0, W1, b1, W2, b2):
    n, din = x.shape
    e = edge_index.shape[1]
    hid = W0.shape[1]
    ncls = W2.shape[1]

    src = edge_index[0]
    dst = edge_index[1]

    deg_parts = _make_deg_kernel(n, e)(src, dst)
    out_deg = (deg_parts[0, :, 0] + deg_parts[1, :, 0]).reshape(n, 1)
    in_deg = (deg_parts[0, :, 64] + deg_parts[1, :, 64]).reshape(n, 1)

    scat_hid = _make_scatter_kernel(n, e, hid)
    src2 = src.reshape(e // _EDGE_BATCH, _EDGE_BATCH)
    dst2 = dst.reshape(e // _EDGE_BATCH, _EDGE_BATCH)

    W2p = jnp.pad(W2, ((0, 0), (0, hid - ncls)))

    h0 = _mm_first(x, out_deg, W0)
    p0 = scat_hid(h0, src2, dst2)
    h1 = _mm_mid(p0[0], p0[1], in_deg, out_deg, b0.reshape(1, -1), W1)
    p1 = scat_hid(h1, src2, dst2)
    h2 = _mm_mid(p1[0], p1[1], in_deg, out_deg, b1.reshape(1, -1), W2p)
    p2 = scat_hid(h2, src2, dst2)
    out = _mm_last(p2[0], p2[1], in_deg, b2.reshape(1, -1))
    return out


# speedup vs baseline: 8.9902x; 1.1010x over previous
import functools

import jax
import jax.numpy as jnp
from jax import lax
from jax.experimental import pallas as pl
from jax.experimental.pallas import tpu as pltpu
from jax.experimental.pallas import tpu_sc as plsc

_NC = 2
_NS = 16
_NW = _NC * _NS
_LANES = 16

_EDGE_BATCH = 128
_ROW_CHUNK = 40


def _zero_vmem_rows(buf, n_rows, row_words):
    nv = row_words // _LANES

    def body(i, c):
        for j in range(nv):
            buf[i, pl.ds(j * _LANES, _LANES)] = jnp.zeros((_LANES,), jnp.float32)
        return c

    lax.fori_loop(0, n_rows, body, 0)


def _fill_vmem_rows(buf, n_rows, row_words, value):
    nv = row_words // _LANES

    def body(i, c):
        for j in range(nv):
            buf[i, pl.ds(j * _LANES, _LANES)] = jnp.full((_LANES,), value, jnp.float32)
        return c

    lax.fori_loop(0, n_rows, body, 0)


def _chunk_loop(n_chunks, sid, fn):
    rounds = (n_chunks + _NS - 1) // _NS

    def body(t, c):
        ch = t * _NS + sid

        @pl.when(ch < n_chunks)
        def _():
            fn(ch)

        return c

    lax.fori_loop(0, rounds, body, 0)


_DEG_W = 128


def _edge_blocks(n_batches):
    per_tile = -(-n_batches // _NW) // 8 * 8 + 8
    last_tile = n_batches - per_tile * (_NW - 1)
    assert 0 < last_tile <= per_tile

    def blocks_of(total):
        out = []
        while total >= 24:
            out.append(24)
            total -= 24
        while total > 0:
            out.append(min(8, total))
            total -= min(8, total)
        assert all(b % 2 == 0 for b in out)
        return out

    return per_tile, blocks_of(per_tile), blocks_of(last_tile)


def _make_deg_kernel(n_nodes, n_edges):
    n_batches = n_edges // _EDGE_BATCH
    per_tile, blocks_main, blocks_last = _edge_blocks(n_batches)
    max_blk = max(blocks_main)
    n_chunks = n_nodes // _ROW_CHUNK
    assert n_nodes % _ROW_CHUNK == 0

    mesh = plsc.VectorSubcoreMesh(core_axis_name="c", subcore_axis_name="s")

    @functools.partial(
        pl.kernel,
        mesh=mesh,
        out_type=jax.ShapeDtypeStruct((_NC, n_nodes, _DEG_W), jnp.float32),
        scratch_types=[
            pltpu.VMEM_SHARED((n_nodes, _DEG_W), jnp.float32),
            pltpu.VMEM((_ROW_CHUNK, _DEG_W), jnp.float32),
            pltpu.VMEM((_EDGE_BATCH, _DEG_W), jnp.float32),
            pltpu.VMEM((_EDGE_BATCH, _DEG_W), jnp.float32),
            pltpu.VMEM((max_blk, _EDGE_BATCH), jnp.int32),
            pltpu.VMEM((max_blk, _EDGE_BATCH), jnp.int32),
            pltpu.SemaphoreType.DMA,
        ],
    )
    def deg_kernel(src_hbm, dst_hbm, out_hbm, acc, bounce, ones_s, ones_d,
                   sblk, dblk, sem):
        cid = lax.axis_index("c")
        sid = lax.axis_index("s")
        wid = sid * _NC + cid

        _zero_vmem_rows(bounce, _ROW_CHUNK, _DEG_W)

        def zero_chunk(ch):
            r0 = pl.multiple_of(ch * _ROW_CHUNK, 8)
            pltpu.sync_copy(bounce, acc.at[pl.ds(r0, _ROW_CHUNK)])

        _chunk_loop(n_chunks, sid, zero_chunk)

        half = _DEG_W // 2
        nv = _DEG_W // _LANES

        def fill(i, c):
            for j in range(nv):
                v = 1.0 if j * _LANES < half else 0.0
                ones_s[i, pl.ds(j * _LANES, _LANES)] = jnp.full(
                    (_LANES,), v, jnp.float32)
                ones_d[i, pl.ds(j * _LANES, _LANES)] = jnp.full(
                    (_LANES,), 1.0 - v, jnp.float32)
            return c

        lax.fori_loop(0, _EDGE_BATCH, fill, 0)
        plsc.subcore_barrier()

        def wait2():
            pltpu.make_async_copy(ones_s, acc.at[sblk.at[0]], sem).wait()
            pltpu.make_async_copy(ones_s, acc.at[sblk.at[0]], sem).wait()

        def run_block(row0, blk):
            pltpu.sync_copy(src_hbm.at[pl.ds(row0, blk)],
                            sblk.at[pl.ds(0, blk)])
            pltpu.sync_copy(dst_hbm.at[pl.ds(row0, blk)],
                            dblk.at[pl.ds(0, blk)])

            def body(j, c):
                pltpu.async_copy(ones_s, acc.at[sblk.at[j]], sem, add=True)
                pltpu.async_copy(ones_d, acc.at[dblk.at[j]], sem, add=True)

                @pl.when(j > 0)
                def _():
                    wait2()

                return c

            lax.fori_loop(0, blk, body, 0)
            wait2()

        @pl.when(wid < _NW - 1)
        def _():
            base = pl.multiple_of(wid * per_tile, 8)
            off = 0
            for blk in blocks_main:
                run_block(base + off, blk)
                off += blk

        @pl.when(wid == _NW - 1)
        def _():
            off = per_tile * (_NW - 1)
            for blk in blocks_last:
                run_block(off, blk)
                off += blk

        plsc.subcore_barrier()

        def drain_chunk(ch):
            r0 = pl.multiple_of(ch * _ROW_CHUNK, 8)
            pltpu.sync_copy(acc.at[pl.ds(r0, _ROW_CHUNK)], bounce)
            pltpu.sync_copy(bounce, out_hbm.at[cid, pl.ds(r0, _ROW_CHUNK)])

        _chunk_loop(n_chunks, sid, drain_chunk)

    return deg_kernel


def _make_scatter_kernel(n_nodes, n_edges, d):
    n_batches = n_edges // _EDGE_BATCH
    assert n_edges % _EDGE_BATCH == 0
    per_tile, blocks_main, blocks_last = _edge_blocks(n_batches)
    max_blk = max(blocks_main)

    n_chunks = n_nodes // _ROW_CHUNK
    assert n_nodes % _ROW_CHUNK == 0

    mesh = plsc.VectorSubcoreMesh(core_axis_name="c", subcore_axis_name="s")

    @functools.partial(
        pl.kernel,
        mesh=mesh,
        out_type=jax.ShapeDtypeStruct((_NC, n_nodes, d), jnp.float32),
        scratch_types=[
            pltpu.VMEM_SHARED((n_nodes, d), jnp.float32),
            pltpu.VMEM((_ROW_CHUNK, d), jnp.float32),
            pltpu.VMEM((_EDGE_BATCH, d), jnp.float32),
            pltpu.VMEM((_EDGE_BATCH, d), jnp.float32),
            pltpu.VMEM((max_blk, _EDGE_BATCH), jnp.int32),
            pltpu.VMEM((max_blk, _EDGE_BATCH), jnp.int32),
            pltpu.SemaphoreType.DMA,
            pltpu.SemaphoreType.DMA,
            pltpu.SemaphoreType.DMA,
            pltpu.SemaphoreType.DMA,
        ],
    )
    def scatter_kernel(h_hbm, src_hbm, dst_hbm, out_hbm, acc, bounce,
                       g0, g1, sblk, dblk, gs0, gs1, ss0, ss1):
        cid = lax.axis_index("c")
        sid = lax.axis_index("s")
        wid = sid * _NC + cid

        _zero_vmem_rows(bounce, _ROW_CHUNK, d)

        def zero_chunk(ch):
            r0 = pl.multiple_of(ch * _ROW_CHUNK, 8)
            pltpu.sync_copy(bounce, acc.at[pl.ds(r0, _ROW_CHUNK)])

        _chunk_loop(n_chunks, sid, zero_chunk)
        plsc.subcore_barrier()

        def run_block(row0, blk):
            pltpu.sync_copy(src_hbm.at[pl.ds(row0, blk)],
                            sblk.at[pl.ds(0, blk)])
            pltpu.sync_copy(dst_hbm.at[pl.ds(row0, blk)],
                            dblk.at[pl.ds(0, blk)])
            pltpu.async_copy(h_hbm.at[sblk.at[0]], g0, gs0)

            def pair(t, c):
                j0 = 2 * t
                pltpu.async_copy(h_hbm.at[sblk.at[j0 + 1]], g1, gs1)
                pltpu.make_async_copy(h_hbm.at[sblk.at[j0]], g0, gs0).wait()
                pltpu.async_copy(g0, acc.at[dblk.at[j0]], ss0, add=True)

                @pl.when(t < blk // 2 - 1)
                def _():
                    pltpu.make_async_copy(
                        g0, acc.at[dblk.at[j0]], ss0).wait()
                    pltpu.async_copy(h_hbm.at[sblk.at[j0 + 2]], g0, gs0)

                pltpu.make_async_copy(h_hbm.at[sblk.at[j0 + 1]], g1, gs1).wait()
                pltpu.async_copy(g1, acc.at[dblk.at[j0 + 1]], ss1, add=True)

                @pl.when(t < blk // 2 - 1)
                def _():
                    pltpu.make_async_copy(
                        g1, acc.at[dblk.at[j0 + 1]], ss1).wait()

                return c

            lax.fori_loop(0, blk // 2, pair, 0)
            pltpu.make_async_copy(g0, acc.at[dblk.at[0]], ss0).wait()
            pltpu.make_async_copy(g1, acc.at[dblk.at[0]], ss1).wait()

        @pl.when(wid < _NW - 1)
        def _():
            base = pl.multiple_of(wid * per_tile, 8)
            off = 0
            for blk in blocks_main:
                run_block(base + off, blk)
                off += blk

        @pl.when(wid == _NW - 1)
        def _():
            off = per_tile * (_NW - 1)
            for blk in blocks_last:
                run_block(off, blk)
                off += blk

        plsc.subcore_barrier()

        def drain_chunk(ch):
            r0 = pl.multiple_of(ch * _ROW_CHUNK, 8)
            pltpu.sync_copy(acc.at[pl.ds(r0, _ROW_CHUNK)], bounce)
            pltpu.sync_copy(bounce, out_hbm.at[cid, pl.ds(r0, _ROW_CHUNK)])

        _chunk_loop(n_chunks, sid, drain_chunk)

    return scatter_kernel


_ROW_BLOCK = 1000


def _mm_first_body(x_ref, deg_ref, w_ref, o_ref):
    onorm = lax.rsqrt(jnp.maximum(deg_ref[...], 1.0))
    o_ref[...] = jnp.dot(x_ref[...] * onorm, w_ref[...],
                         preferred_element_type=jnp.float32,
                         precision=lax.Precision.HIGHEST)


def _mm_first(x, out_deg, w):
    n, din = x.shape
    dout = w.shape[1]
    grid = (n // _ROW_BLOCK,)
    return pl.pallas_call(
        _mm_first_body,
        grid=grid,
        in_specs=[
            pl.BlockSpec((_ROW_BLOCK, din), lambda i: (i, 0)),
            pl.BlockSpec((_ROW_BLOCK, 1), lambda i: (i, 0)),
            pl.BlockSpec((din, dout), lambda i: (0, 0)),
        ],
        out_specs=pl.BlockSpec((_ROW_BLOCK, dout), lambda i: (i, 0)),
        out_shape=jax.ShapeDtypeStruct((n, dout), jnp.float32),
    )(x, out_deg, w)


def _mm_mid_body(a0_ref, a1_ref, ideg_ref, odeg_ref, b_ref, w_ref, o_ref):
    inorm = lax.rsqrt(jnp.maximum(ideg_ref[...], 1.0))
    onorm = lax.rsqrt(jnp.maximum(odeg_ref[...], 1.0))
    rst = (a0_ref[...] + a1_ref[...]) * inorm + b_ref[...]
    rst = jnp.maximum(rst, 0.0)
    o_ref[...] = jnp.dot(rst * onorm, w_ref[...],
                         preferred_element_type=jnp.float32,
                         precision=lax.Precision.HIGHEST)


def _mm_mid(a0, a1, in_deg, out_deg, b, w):
    n, din = a0.shape
    dout = w.shape[1]
    grid = (n // _ROW_BLOCK,)
    return pl.pallas_call(
        _mm_mid_body,
        grid=grid,
        in_specs=[
            pl.BlockSpec((_ROW_BLOCK, din), lambda i: (i, 0)),
            pl.BlockSpec((_ROW_BLOCK, din), lambda i: (i, 0)),
            pl.BlockSpec((_ROW_BLOCK, 1), lambda i: (i, 0)),
            pl.BlockSpec((_ROW_BLOCK, 1), lambda i: (i, 0)),
            pl.BlockSpec((1, din), lambda i: (0, 0)),
            pl.BlockSpec((din, dout), lambda i: (0, 0)),
        ],
        out_specs=pl.BlockSpec((_ROW_BLOCK, dout), lambda i: (i, 0)),
        out_shape=jax.ShapeDtypeStruct((n, dout), jnp.float32),
    )(a0, a1, in_deg, out_deg, b, w)


def _mm_last_body(dout, a0_ref, a1_ref, ideg_ref, b_ref, o_ref):
    inorm = lax.rsqrt(jnp.maximum(ideg_ref[...], 1.0))
    agg = (a0_ref[...] + a1_ref[...])[:, :dout]
    o_ref[...] = agg * inorm + b_ref[...]


def _mm_last(a0, a1, in_deg, b):
    n, dpad = a0.shape
    dout = b.shape[1]
    grid = (n // _ROW_BLOCK,)
    return pl.pallas_call(
        functools.partial(_mm_last_body, dout),
        grid=grid,
        in_specs=[
            pl.BlockSpec((_ROW_BLOCK, dpad), lambda i: (i, 0)),
            pl.BlockSpec((_ROW_BLOCK, dpad), lambda i: (i, 0)),
            pl.BlockSpec((_ROW_BLOCK, 1), lambda i: (i, 0)),
            pl.BlockSpec((1, dout), lambda i: (0, 0)),
        ],
        out_specs=pl.BlockSpec((_ROW_BLOCK, dout), lambda i: (i, 0)),
        out_shape=jax.ShapeDtypeStruct((n, dout), jnp.float32),
    )(a0, a1, in_deg, b)


def kernel(x, edge_index, W0, b0, W1, b1, W2, b2):
    n, din = x.shape
    e = edge_index.shape[1]
    hid = W0.shape[1]
    ncls = W2.shape[1]

    src2 = edge_index[0].reshape(e // _EDGE_BATCH, _EDGE_BATCH)
    dst2 = edge_index[1].reshape(e // _EDGE_BATCH, _EDGE_BATCH)

    deg_parts = _make_deg_kernel(n, e)(src2, dst2)
    out_deg = (deg_parts[0, :, 0] + deg_parts[1, :, 0]).reshape(n, 1)
    in_deg = (deg_parts[0, :, 64] + deg_parts[1, :, 64]).reshape(n, 1)

    scat_hid = _make_scatter_kernel(n, e, hid)

    W2p = jnp.pad(W2, ((0, 0), (0, hid - ncls)))

    h0 = _mm_first(x, out_deg, W0)
    p0 = scat_hid(h0, src2, dst2)
    h1 = _mm_mid(p0[0], p0[1], in_deg, out_deg, b0.reshape(1, -1), W1)
    p1 = scat_hid(h1, src2, dst2)
    h2 = _mm_mid(p1[0], p1[1], in_deg, out_deg, b1.reshape(1, -1), W2p)
    p2 = scat_hid(h2, src2, dst2)
    out = _mm_last(p2[0], p2[1], in_deg, b2.reshape(1, -1))
    return out


# speedup vs baseline: 9.1885x; 1.0221x over previous
import functools

import jax
import jax.numpy as jnp
from jax import lax
from jax.experimental import pallas as pl
from jax.experimental.pallas import tpu as pltpu
from jax.experimental.pallas import tpu_sc as plsc

_NC = 2
_NS = 16
_NW = _NC * _NS
_LANES = 16

_EDGE_BATCH = 128
_ROW_CHUNK = 40


def _zero_vmem_rows(buf, n_rows, row_words):
    nv = row_words // _LANES

    def body(i, c):
        for j in range(nv):
            buf[i, pl.ds(j * _LANES, _LANES)] = jnp.zeros((_LANES,), jnp.float32)
        return c

    lax.fori_loop(0, n_rows, body, 0)


def _fill_vmem_rows(buf, n_rows, row_words, value):
    nv = row_words // _LANES

    def body(i, c):
        for j in range(nv):
            buf[i, pl.ds(j * _LANES, _LANES)] = jnp.full((_LANES,), value, jnp.float32)
        return c

    lax.fori_loop(0, n_rows, body, 0)


def _chunk_loop(n_chunks, sid, fn):
    rounds = (n_chunks + _NS - 1) // _NS

    def body(t, c):
        ch = t * _NS + sid

        @pl.when(ch < n_chunks)
        def _():
            fn(ch)

        return c

    lax.fori_loop(0, rounds, body, 0)


_DEG_W = 128


def _edge_blocks(n_batches):
    per_tile = -(-n_batches // _NW) // 8 * 8 + 8
    last_tile = n_batches - per_tile * (_NW - 1)
    assert 0 < last_tile <= per_tile

    def blocks_of(total):
        out = []
        while total >= 24:
            out.append(24)
            total -= 24
        while total > 0:
            out.append(min(8, total))
            total -= min(8, total)
        assert all(b % 2 == 0 for b in out)
        return out

    return per_tile, blocks_of(per_tile), blocks_of(last_tile)


def _make_deg_kernel(n_nodes, n_edges):
    n_batches = n_edges // _EDGE_BATCH
    per_tile, blocks_main, blocks_last = _edge_blocks(n_batches)
    max_blk = max(blocks_main)
    n_chunks = n_nodes // _ROW_CHUNK
    assert n_nodes % _ROW_CHUNK == 0

    mesh = plsc.VectorSubcoreMesh(core_axis_name="c", subcore_axis_name="s")

    @functools.partial(
        pl.kernel,
        mesh=mesh,
        out_type=jax.ShapeDtypeStruct((_NC, n_nodes, _DEG_W), jnp.float32),
        scratch_types=[
            pltpu.VMEM_SHARED((n_nodes, _DEG_W), jnp.float32),
            pltpu.VMEM((_ROW_CHUNK, _DEG_W), jnp.float32),
            pltpu.VMEM((_EDGE_BATCH, _DEG_W), jnp.float32),
            pltpu.VMEM((_EDGE_BATCH, _DEG_W), jnp.float32),
            pltpu.VMEM((max_blk, _EDGE_BATCH), jnp.int32),
            pltpu.VMEM((max_blk, _EDGE_BATCH), jnp.int32),
            pltpu.SemaphoreType.DMA,
        ],
    )
    def deg_kernel(src_hbm, dst_hbm, out_hbm, acc, bounce, ones_s, ones_d,
                   sblk, dblk, sem):
        cid = lax.axis_index("c")
        sid = lax.axis_index("s")
        wid = sid * _NC + cid

        _zero_vmem_rows(bounce, _ROW_CHUNK, _DEG_W)

        def zero_chunk(ch):
            r0 = pl.multiple_of(ch * _ROW_CHUNK, 8)
            pltpu.sync_copy(bounce, acc.at[pl.ds(r0, _ROW_CHUNK)])

        _chunk_loop(n_chunks, sid, zero_chunk)

        half = _DEG_W // 2
        nv = _DEG_W // _LANES

        def fill(i, c):
            for j in range(nv):
                v = 1.0 if j * _LANES < half else 0.0
                ones_s[i, pl.ds(j * _LANES, _LANES)] = jnp.full(
                    (_LANES,), v, jnp.float32)
                ones_d[i, pl.ds(j * _LANES, _LANES)] = jnp.full(
                    (_LANES,), 1.0 - v, jnp.float32)
            return c

        lax.fori_loop(0, _EDGE_BATCH, fill, 0)
        plsc.subcore_barrier()

        def wait2():
            pltpu.make_async_copy(ones_s, acc.at[sblk.at[0]], sem).wait()
            pltpu.make_async_copy(ones_s, acc.at[sblk.at[0]], sem).wait()

        def run_block(row0, blk):
            pltpu.sync_copy(src_hbm.at[pl.ds(row0, blk)],
                            sblk.at[pl.ds(0, blk)])
            pltpu.sync_copy(dst_hbm.at[pl.ds(row0, blk)],
                            dblk.at[pl.ds(0, blk)])

            def body(j, c):
                pltpu.async_copy(ones_s, acc.at[sblk.at[j]], sem, add=True)
                pltpu.async_copy(ones_d, acc.at[dblk.at[j]], sem, add=True)

                @pl.when(j > 0)
                def _():
                    wait2()

                return c

            lax.fori_loop(0, blk, body, 0)
            wait2()

        @pl.when(wid < _NW - 1)
        def _():
            base = pl.multiple_of(wid * per_tile, 8)
            off = 0
            for blk in blocks_main:
                run_block(base + off, blk)
                off += blk

        @pl.when(wid == _NW - 1)
        def _():
            off = per_tile * (_NW - 1)
            for blk in blocks_last:
                run_block(off, blk)
                off += blk

        plsc.subcore_barrier()

        def drain_chunk(ch):
            r0 = pl.multiple_of(ch * _ROW_CHUNK, 8)
            pltpu.sync_copy(acc.at[pl.ds(r0, _ROW_CHUNK)], bounce)
            pltpu.sync_copy(bounce, out_hbm.at[cid, pl.ds(r0, _ROW_CHUNK)])

        _chunk_loop(n_chunks, sid, drain_chunk)

    return deg_kernel


def _make_scatter_kernel(n_nodes, n_edges, d):
    n_batches = n_edges // _EDGE_BATCH
    assert n_edges % _EDGE_BATCH == 0
    per_tile, blocks_main, blocks_last = _edge_blocks(n_batches)
    max_blk = max(blocks_main)

    n_chunks = n_nodes // _ROW_CHUNK
    assert n_nodes % _ROW_CHUNK == 0

    mesh = plsc.VectorSubcoreMesh(core_axis_name="c", subcore_axis_name="s")

    @functools.partial(
        pl.kernel,
        mesh=mesh,
        out_type=jax.ShapeDtypeStruct((_NC, n_nodes, d), jnp.float32),
        scratch_types=[
            pltpu.VMEM_SHARED((n_nodes, d), jnp.float32),
            pltpu.VMEM((_ROW_CHUNK, d), jnp.float32),
            pltpu.VMEM((_EDGE_BATCH, d), jnp.float32),
            pltpu.VMEM((_EDGE_BATCH, d), jnp.float32),
            pltpu.VMEM((max_blk, _EDGE_BATCH), jnp.int32),
            pltpu.VMEM((max_blk, _EDGE_BATCH), jnp.int32),
            pltpu.SemaphoreType.DMA,
            pltpu.SemaphoreType.DMA,
            pltpu.SemaphoreType.DMA,
            pltpu.SemaphoreType.DMA,
        ],
    )
    def scatter_kernel(h_hbm, src_hbm, dst_hbm, out_hbm, acc, bounce,
                       g0, g1, sblk, dblk, gs0, gs1, ss0, ss1):
        cid = lax.axis_index("c")
        sid = lax.axis_index("s")
        wid = sid * _NC + cid

        _zero_vmem_rows(bounce, _ROW_CHUNK, d)

        def zero_chunk(ch):
            r0 = pl.multiple_of(ch * _ROW_CHUNK, 8)
            pltpu.sync_copy(bounce, acc.at[pl.ds(r0, _ROW_CHUNK)])

        _chunk_loop(n_chunks, sid, zero_chunk)
        plsc.subcore_barrier()

        def run_block(row0, blk):
            pltpu.sync_copy(src_hbm.at[pl.ds(row0, blk)],
                            sblk.at[pl.ds(0, blk)])
            pltpu.sync_copy(dst_hbm.at[pl.ds(row0, blk)],
                            dblk.at[pl.ds(0, blk)])
            pltpu.async_copy(h_hbm.at[sblk.at[0]], g0, gs0)

            def pair(t, c):
                j0 = 2 * t
                pltpu.async_copy(h_hbm.at[sblk.at[j0 + 1]], g1, gs1)
                pltpu.make_async_copy(h_hbm.at[sblk.at[j0]], g0, gs0).wait()
                pltpu.async_copy(g0, acc.at[dblk.at[j0]], ss0, add=True)

                @pl.when(t < blk // 2 - 1)
                def _():
                    pltpu.make_async_copy(
                        g0, acc.at[dblk.at[j0]], ss0).wait()
                    pltpu.async_copy(h_hbm.at[sblk.at[j0 + 2]], g0, gs0)

                pltpu.make_async_copy(h_hbm.at[sblk.at[j0 + 1]], g1, gs1).wait()
                pltpu.async_copy(g1, acc.at[dblk.at[j0 + 1]], ss1, add=True)

                @pl.when(t < blk // 2 - 1)
                def _():
                    pltpu.make_async_copy(
                        g1, acc.at[dblk.at[j0 + 1]], ss1).wait()

                return c

            lax.fori_loop(0, blk // 2, pair, 0)
            pltpu.make_async_copy(g0, acc.at[dblk.at[0]], ss0).wait()
            pltpu.make_async_copy(g1, acc.at[dblk.at[0]], ss1).wait()

        @pl.when(wid < _NW - 1)
        def _():
            base = pl.multiple_of(wid * per_tile, 8)
            off = 0
            for blk in blocks_main:
                run_block(base + off, blk)
                off += blk

        @pl.when(wid == _NW - 1)
        def _():
            off = per_tile * (_NW - 1)
            for blk in blocks_last:
                run_block(off, blk)
                off += blk

        plsc.subcore_barrier()

        def drain_chunk(ch):
            r0 = pl.multiple_of(ch * _ROW_CHUNK, 8)
            pltpu.sync_copy(acc.at[pl.ds(r0, _ROW_CHUNK)], bounce)
            pltpu.sync_copy(bounce, out_hbm.at[cid, pl.ds(r0, _ROW_CHUNK)])

        _chunk_loop(n_chunks, sid, drain_chunk)

    return scatter_kernel


_ROW_BLOCK = 2000


def _mm_first_body(x_ref, deg_ref, w_ref, o_ref):
    onorm = lax.rsqrt(jnp.maximum(deg_ref[...], 1.0))
    o_ref[...] = jnp.dot(x_ref[...] * onorm, w_ref[...],
                         preferred_element_type=jnp.float32,
                         precision=lax.Precision.HIGHEST)


def _mm_first(x, out_deg, w):
    n, din = x.shape
    dout = w.shape[1]
    grid = (n // _ROW_BLOCK,)
    return pl.pallas_call(
        _mm_first_body,
        grid=grid,
        in_specs=[
            pl.BlockSpec((_ROW_BLOCK, din), lambda i: (i, 0)),
            pl.BlockSpec((_ROW_BLOCK, 1), lambda i: (i, 0)),
            pl.BlockSpec((din, dout), lambda i: (0, 0)),
        ],
        out_specs=pl.BlockSpec((_ROW_BLOCK, dout), lambda i: (i, 0)),
        out_shape=jax.ShapeDtypeStruct((n, dout), jnp.float32),
    )(x, out_deg, w)


def _mm_mid_body(a0_ref, a1_ref, ideg_ref, odeg_ref, b_ref, w_ref, o_ref):
    inorm = lax.rsqrt(jnp.maximum(ideg_ref[...], 1.0))
    onorm = lax.rsqrt(jnp.maximum(odeg_ref[...], 1.0))
    rst = (a0_ref[...] + a1_ref[...]) * inorm + b_ref[...]
    rst = jnp.maximum(rst, 0.0)
    o_ref[...] = jnp.dot(rst * onorm, w_ref[...],
                         preferred_element_type=jnp.float32,
                         precision=lax.Precision.HIGHEST)


def _mm_mid(a0, a1, in_deg, out_deg, b, w):
    n, din = a0.shape
    dout = w.shape[1]
    grid = (n // _ROW_BLOCK,)
    return pl.pallas_call(
        _mm_mid_body,
        grid=grid,
        in_specs=[
            pl.BlockSpec((_ROW_BLOCK, din), lambda i: (i, 0)),
            pl.BlockSpec((_ROW_BLOCK, din), lambda i: (i, 0)),
            pl.BlockSpec((_ROW_BLOCK, 1), lambda i: (i, 0)),
            pl.BlockSpec((_ROW_BLOCK, 1), lambda i: (i, 0)),
            pl.BlockSpec((1, din), lambda i: (0, 0)),
            pl.BlockSpec((din, dout), lambda i: (0, 0)),
        ],
        out_specs=pl.BlockSpec((_ROW_BLOCK, dout), lambda i: (i, 0)),
        out_shape=jax.ShapeDtypeStruct((n, dout), jnp.float32),
    )(a0, a1, in_deg, out_deg, b, w)


def _mm_last_body(dout, a0_ref, a1_ref, ideg_ref, b_ref, o_ref):
    inorm = lax.rsqrt(jnp.maximum(ideg_ref[...], 1.0))
    agg = (a0_ref[...] + a1_ref[...])[:, :dout]
    o_ref[...] = agg * inorm + b_ref[...]


def _mm_last(a0, a1, in_deg, b):
    n, dpad = a0.shape
    dout = b.shape[1]
    grid = (n // _ROW_BLOCK,)
    return pl.pallas_call(
        functools.partial(_mm_last_body, dout),
        grid=grid,
        in_specs=[
            pl.BlockSpec((_ROW_BLOCK, dpad), lambda i: (i, 0)),
            pl.BlockSpec((_ROW_BLOCK, dpad), lambda i: (i, 0)),
            pl.BlockSpec((_ROW_BLOCK, 1), lambda i: (i, 0)),
            pl.BlockSpec((1, dout), lambda i: (0, 0)),
        ],
        out_specs=pl.BlockSpec((_ROW_BLOCK, dout), lambda i: (i, 0)),
        out_shape=jax.ShapeDtypeStruct((n, dout), jnp.float32),
    )(a0, a1, in_deg, b)


def kernel(x, edge_index, W0, b0, W1, b1, W2, b2):
    n, din = x.shape
    e = edge_index.shape[1]
    hid = W0.shape[1]
    ncls = W2.shape[1]

    src2 = edge_index[0].reshape(e // _EDGE_BATCH, _EDGE_BATCH)
    dst2 = edge_index[1].reshape(e // _EDGE_BATCH, _EDGE_BATCH)

    deg_parts = _make_deg_kernel(n, e)(src2, dst2)
    out_deg = (deg_parts[0, :, 0] + deg_parts[1, :, 0]).reshape(n, 1)
    in_deg = (deg_parts[0, :, 64] + deg_parts[1, :, 64]).reshape(n, 1)

    scat_hid = _make_scatter_kernel(n, e, hid)

    W2p = jnp.pad(W2, ((0, 0), (0, hid - ncls)))

    h0 = _mm_first(x, out_deg, W0)
    p0 = scat_hid(h0, src2, dst2)
    h1 = _mm_mid(p0[0], p0[1], in_deg, out_deg, b0.reshape(1, -1), W1)
    p1 = scat_hid(h1, src2, dst2)
    h2 = _mm_mid(p1[0], p1[1], in_deg, out_deg, b1.reshape(1, -1), W2p)
    p2 = scat_hid(h2, src2, dst2)
    out = _mm_last(p2[0], p2[1], in_deg, b2.reshape(1, -1))
    return out


# speedup vs baseline: 9.2329x; 1.0048x over previous
import functools

import jax
import jax.numpy as jnp
from jax import lax
from jax.experimental import pallas as pl
from jax.experimental.pallas import tpu as pltpu
from jax.experimental.pallas import tpu_sc as plsc

_NC = 2
_NS = 16
_NW = _NC * _NS
_LANES = 16

_EDGE_BATCH = 128
_ROW_CHUNK = 40


def _zero_vmem_rows(buf, n_rows, row_words):
    nv = row_words // _LANES

    def body(i, c):
        for j in range(nv):
            buf[i, pl.ds(j * _LANES, _LANES)] = jnp.zeros((_LANES,), jnp.float32)
        return c

    lax.fori_loop(0, n_rows, body, 0)


def _fill_vmem_rows(buf, n_rows, row_words, value):
    nv = row_words // _LANES

    def body(i, c):
        for j in range(nv):
            buf[i, pl.ds(j * _LANES, _LANES)] = jnp.full((_LANES,), value, jnp.float32)
        return c

    lax.fori_loop(0, n_rows, body, 0)


def _chunk_loop(n_chunks, sid, fn):
    rounds = (n_chunks + _NS - 1) // _NS

    def body(t, c):
        ch = t * _NS + sid

        @pl.when(ch < n_chunks)
        def _():
            fn(ch)

        return c

    lax.fori_loop(0, rounds, body, 0)


_DEG_W = 128


def _edge_blocks(n_batches):
    per_tile = -(-n_batches // _NW) // 8 * 8 + 8
    last_tile = n_batches - per_tile * (_NW - 1)
    assert 0 < last_tile <= per_tile

    def blocks_of(total):
        out = []
        while total >= 24:
            out.append(24)
            total -= 24
        while total > 0:
            out.append(min(8, total))
            total -= min(8, total)
        assert all(b % 2 == 0 for b in out)
        return out

    return per_tile, blocks_of(per_tile), blocks_of(last_tile)


def _make_deg_kernel(n_nodes, n_edges):
    n_batches = n_edges // _EDGE_BATCH
    per_tile, blocks_main, blocks_last = _edge_blocks(n_batches)
    max_blk = max(blocks_main)
    n_chunks = n_nodes // _ROW_CHUNK
    assert n_nodes % _ROW_CHUNK == 0

    mesh = plsc.VectorSubcoreMesh(core_axis_name="c", subcore_axis_name="s")

    @functools.partial(
        pl.kernel,
        mesh=mesh,
        out_type=jax.ShapeDtypeStruct((_NC, n_nodes, _DEG_W), jnp.float32),
        scratch_types=[
            pltpu.VMEM_SHARED((n_nodes, _DEG_W), jnp.float32),
            pltpu.VMEM((_ROW_CHUNK, _DEG_W), jnp.float32),
            pltpu.VMEM((_EDGE_BATCH, _DEG_W), jnp.float32),
            pltpu.VMEM((_EDGE_BATCH, _DEG_W), jnp.float32),
            pltpu.VMEM((max_blk, _EDGE_BATCH), jnp.int32),
            pltpu.VMEM((max_blk, _EDGE_BATCH), jnp.int32),
            pltpu.SemaphoreType.DMA,
        ],
    )
    def deg_kernel(src_hbm, dst_hbm, out_hbm, acc, bounce, ones_s, ones_d,
                   sblk, dblk, sem):
        cid = lax.axis_index("c")
        sid = lax.axis_index("s")
        wid = sid * _NC + cid

        _zero_vmem_rows(bounce, _ROW_CHUNK, _DEG_W)

        def zero_chunk(ch):
            r0 = pl.multiple_of(ch * _ROW_CHUNK, 8)
            pltpu.sync_copy(bounce, acc.at[pl.ds(r0, _ROW_CHUNK)])

        _chunk_loop(n_chunks, sid, zero_chunk)

        half = _DEG_W // 2
        nv = _DEG_W // _LANES

        def fill(i, c):
            for j in range(nv):
                v = 1.0 if j * _LANES < half else 0.0
                ones_s[i, pl.ds(j * _LANES, _LANES)] = jnp.full(
                    (_LANES,), v, jnp.float32)
                ones_d[i, pl.ds(j * _LANES, _LANES)] = jnp.full(
                    (_LANES,), 1.0 - v, jnp.float32)
            return c

        lax.fori_loop(0, _EDGE_BATCH, fill, 0)
        plsc.subcore_barrier()

        def wait2():
            pltpu.make_async_copy(ones_s, acc.at[sblk.at[0]], sem).wait()
            pltpu.make_async_copy(ones_s, acc.at[sblk.at[0]], sem).wait()

        def run_block(row0, blk):
            pltpu.sync_copy(src_hbm.at[pl.ds(row0, blk)],
                            sblk.at[pl.ds(0, blk)])
            pltpu.sync_copy(dst_hbm.at[pl.ds(row0, blk)],
                            dblk.at[pl.ds(0, blk)])

            def body(j, c):
                pltpu.async_copy(ones_s, acc.at[sblk.at[j]], sem, add=True)
                pltpu.async_copy(ones_d, acc.at[dblk.at[j]], sem, add=True)

                @pl.when(j > 0)
                def _():
                    wait2()

                return c

            lax.fori_loop(0, blk, body, 0)
            wait2()

        @pl.when(wid < _NW - 1)
        def _():
            base = pl.multiple_of(wid * per_tile, 8)
            off = 0
            for blk in blocks_main:
                run_block(base + off, blk)
                off += blk

        @pl.when(wid == _NW - 1)
        def _():
            off = per_tile * (_NW - 1)
            for blk in blocks_last:
                run_block(off, blk)
                off += blk

        plsc.subcore_barrier()

        def drain_chunk(ch):
            r0 = pl.multiple_of(ch * _ROW_CHUNK, 8)
            pltpu.sync_copy(acc.at[pl.ds(r0, _ROW_CHUNK)], bounce)
            pltpu.sync_copy(bounce, out_hbm.at[cid, pl.ds(r0, _ROW_CHUNK)])

        _chunk_loop(n_chunks, sid, drain_chunk)

    return deg_kernel


def _make_scatter_kernel(n_nodes, n_edges, d):
    n_batches = n_edges // _EDGE_BATCH
    assert n_edges % _EDGE_BATCH == 0
    per_tile, blocks_main, blocks_last = _edge_blocks(n_batches)
    max_blk = max(blocks_main)

    n_chunks = n_nodes // _ROW_CHUNK
    assert n_nodes % _ROW_CHUNK == 0

    mesh = plsc.VectorSubcoreMesh(core_axis_name="c", subcore_axis_name="s")

    @functools.partial(
        pl.kernel,
        mesh=mesh,
        out_type=jax.ShapeDtypeStruct((_NC, n_nodes, d), jnp.float32),
        scratch_types=[
            pltpu.VMEM_SHARED((n_nodes, d), jnp.float32),
            pltpu.VMEM((_ROW_CHUNK, d), jnp.float32),
            pltpu.VMEM((_EDGE_BATCH, d), jnp.float32),
            pltpu.VMEM((_EDGE_BATCH, d), jnp.float32),
            pltpu.VMEM((max_blk, _EDGE_BATCH), jnp.int32),
            pltpu.VMEM((max_blk, _EDGE_BATCH), jnp.int32),
            pltpu.SemaphoreType.DMA,
            pltpu.SemaphoreType.DMA,
            pltpu.SemaphoreType.DMA,
            pltpu.SemaphoreType.DMA,
        ],
    )
    def scatter_kernel(h_hbm, src_hbm, dst_hbm, out_hbm, acc, bounce,
                       g0, g1, sblk, dblk, gs0, gs1, ss0, ss1):
        cid = lax.axis_index("c")
        sid = lax.axis_index("s")
        wid = sid * _NC + cid

        _zero_vmem_rows(bounce, _ROW_CHUNK, d)

        def zero_chunk(ch):
            r0 = pl.multiple_of(ch * _ROW_CHUNK, 8)
            pltpu.sync_copy(bounce, acc.at[pl.ds(r0, _ROW_CHUNK)])

        _chunk_loop(n_chunks, sid, zero_chunk)
        plsc.subcore_barrier()

        def run_block(row0, blk):
            pltpu.sync_copy(src_hbm.at[pl.ds(row0, blk)],
                            sblk.at[pl.ds(0, blk)])
            pltpu.sync_copy(dst_hbm.at[pl.ds(row0, blk)],
                            dblk.at[pl.ds(0, blk)])
            pltpu.async_copy(h_hbm.at[sblk.at[0]], g0, gs0)

            def pair(t, c):
                j0 = 2 * t
                pltpu.async_copy(h_hbm.at[sblk.at[j0 + 1]], g1, gs1)
                pltpu.make_async_copy(h_hbm.at[sblk.at[j0]], g0, gs0).wait()
                pltpu.async_copy(g0, acc.at[dblk.at[j0]], ss0, add=True)

                @pl.when(t < blk // 2 - 1)
                def _():
                    pltpu.make_async_copy(
                        g0, acc.at[dblk.at[j0]], ss0).wait()
                    pltpu.async_copy(h_hbm.at[sblk.at[j0 + 2]], g0, gs0)

                pltpu.make_async_copy(h_hbm.at[sblk.at[j0 + 1]], g1, gs1).wait()
                pltpu.async_copy(g1, acc.at[dblk.at[j0 + 1]], ss1, add=True)

                @pl.when(t < blk // 2 - 1)
                def _():
                    pltpu.make_async_copy(
                        g1, acc.at[dblk.at[j0 + 1]], ss1).wait()

                return c

            lax.fori_loop(0, blk // 2, pair, 0)
            pltpu.make_async_copy(g0, acc.at[dblk.at[0]], ss0).wait()
            pltpu.make_async_copy(g1, acc.at[dblk.at[0]], ss1).wait()

        @pl.when(wid < _NW - 1)
        def _():
            base = pl.multiple_of(wid * per_tile, 8)
            off = 0
            for blk in blocks_main:
                run_block(base + off, blk)
                off += blk

        @pl.when(wid == _NW - 1)
        def _():
            off = per_tile * (_NW - 1)
            for blk in blocks_last:
                run_block(off, blk)
                off += blk

        plsc.subcore_barrier()

        def drain_chunk(ch):
            r0 = pl.multiple_of(ch * _ROW_CHUNK, 8)
            pltpu.sync_copy(acc.at[pl.ds(r0, _ROW_CHUNK)], bounce)
            pltpu.sync_copy(bounce, out_hbm.at[cid, pl.ds(r0, _ROW_CHUNK)])

        _chunk_loop(n_chunks, sid, drain_chunk)

    return scatter_kernel


_ROW_BLOCK = 2000


def _mm_plain_body(x_ref, w_ref, o_ref):
    o_ref[...] = jnp.dot(x_ref[...], w_ref[...],
                         preferred_element_type=jnp.float32,
                         precision=lax.Precision.HIGHEST)


def _mm_plain(x, w):
    n, din = x.shape
    dout = w.shape[1]
    grid = (n // _ROW_BLOCK,)
    return pl.pallas_call(
        _mm_plain_body,
        grid=grid,
        in_specs=[
            pl.BlockSpec((_ROW_BLOCK, din), lambda i: (i, 0)),
            pl.BlockSpec((din, dout), lambda i: (0, 0)),
        ],
        out_specs=pl.BlockSpec((_ROW_BLOCK, dout), lambda i: (i, 0)),
        out_shape=jax.ShapeDtypeStruct((n, dout), jnp.float32),
    )(x, w)


def _scale_body(a_ref, deg_ref, o_ref):
    onorm = lax.rsqrt(jnp.maximum(deg_ref[...], 1.0))
    o_ref[...] = a_ref[...] * onorm


def _scale_rows(a, out_deg):
    n, d = a.shape
    grid = (n // _ROW_BLOCK,)
    return pl.pallas_call(
        _scale_body,
        grid=grid,
        in_specs=[
            pl.BlockSpec((_ROW_BLOCK, d), lambda i: (i, 0)),
            pl.BlockSpec((_ROW_BLOCK, 1), lambda i: (i, 0)),
        ],
        out_specs=pl.BlockSpec((_ROW_BLOCK, d), lambda i: (i, 0)),
        out_shape=jax.ShapeDtypeStruct((n, d), jnp.float32),
    )(a, out_deg)


def _mm_mid_body(a0_ref, a1_ref, ideg_ref, odeg_ref, b_ref, w_ref, o_ref):
    inorm = lax.rsqrt(jnp.maximum(ideg_ref[...], 1.0))
    onorm = lax.rsqrt(jnp.maximum(odeg_ref[...], 1.0))
    rst = (a0_ref[...] + a1_ref[...]) * inorm + b_ref[...]
    rst = jnp.maximum(rst, 0.0)
    o_ref[...] = jnp.dot(rst * onorm, w_ref[...],
                         preferred_element_type=jnp.float32,
                         precision=lax.Precision.HIGHEST)


def _mm_mid(a0, a1, in_deg, out_deg, b, w):
    n, din = a0.shape
    dout = w.shape[1]
    grid = (n // _ROW_BLOCK,)
    return pl.pallas_call(
        _mm_mid_body,
        grid=grid,
        in_specs=[
            pl.BlockSpec((_ROW_BLOCK, din), lambda i: (i, 0)),
            pl.BlockSpec((_ROW_BLOCK, din), lambda i: (i, 0)),
            pl.BlockSpec((_ROW_BLOCK, 1), lambda i: (i, 0)),
            pl.BlockSpec((_ROW_BLOCK, 1), lambda i: (i, 0)),
            pl.BlockSpec((1, din), lambda i: (0, 0)),
            pl.BlockSpec((din, dout), lambda i: (0, 0)),
        ],
        out_specs=pl.BlockSpec((_ROW_BLOCK, dout), lambda i: (i, 0)),
        out_shape=jax.ShapeDtypeStruct((n, dout), jnp.float32),
    )(a0, a1, in_deg, out_deg, b, w)


def _mm_last_body(dout, a0_ref, a1_ref, ideg_ref, b_ref, o_ref):
    inorm = lax.rsqrt(jnp.maximum(ideg_ref[...], 1.0))
    agg = (a0_ref[...] + a1_ref[...])[:, :dout]
    o_ref[...] = agg * inorm + b_ref[...]


def _mm_last(a0, a1, in_deg, b):
    n, dpad = a0.shape
    dout = b.shape[1]
    grid = (n // _ROW_BLOCK,)
    return pl.pallas_call(
        functools.partial(_mm_last_body, dout),
        grid=grid,
        in_specs=[
            pl.BlockSpec((_ROW_BLOCK, dpad), lambda i: (i, 0)),
            pl.BlockSpec((_ROW_BLOCK, dpad), lambda i: (i, 0)),
            pl.BlockSpec((_ROW_BLOCK, 1), lambda i: (i, 0)),
            pl.BlockSpec((1, dout), lambda i: (0, 0)),
        ],
        out_specs=pl.BlockSpec((_ROW_BLOCK, dout), lambda i: (i, 0)),
        out_shape=jax.ShapeDtypeStruct((n, dout), jnp.float32),
    )(a0, a1, in_deg, b)


def kernel(x, edge_index, W0, b0, W1, b1, W2, b2):
    n, din = x.shape
    e = edge_index.shape[1]
    hid = W0.shape[1]
    ncls = W2.shape[1]

    src2 = edge_index[0].reshape(e // _EDGE_BATCH, _EDGE_BATCH)
    dst2 = edge_index[1].reshape(e // _EDGE_BATCH, _EDGE_BATCH)

    mm0 = _mm_plain(x, W0)

    deg_parts = _make_deg_kernel(n, e)(src2, dst2)
    out_deg = (deg_parts[0, :, 0] + deg_parts[1, :, 0]).reshape(n, 1)
    in_deg = (deg_parts[0, :, _DEG_W // 2]
              + deg_parts[1, :, _DEG_W // 2]).reshape(n, 1)

    scat_hid = _make_scatter_kernel(n, e, hid)

    W2p = jnp.pad(W2, ((0, 0), (0, hid - ncls)))

    h0 = _scale_rows(mm0, out_deg)
    p0 = scat_hid(h0, src2, dst2)
    h1 = _mm_mid(p0[0], p0[1], in_deg, out_deg, b0.reshape(1, -1), W1)
    p1 = scat_hid(h1, src2, dst2)
    h2 = _mm_mid(p1[0], p1[1], in_deg, out_deg, b1.reshape(1, -1), W2p)
    p2 = scat_hid(h2, src2, dst2)
    out = _mm_last(p2[0], p2[1], in_deg, b2.reshape(1, -1))
    return out


# speedup vs baseline: 9.2340x; 1.0001x over previous
import functools

import jax
import jax.numpy as jnp
from jax import lax
from jax.experimental import pallas as pl
from jax.experimental.pallas import tpu as pltpu
from jax.experimental.pallas import tpu_sc as plsc

_NC = 2
_NS = 16
_NW = _NC * _NS
_LANES = 16

_EDGE_BATCH = 128
_ROW_CHUNK = 40


def _zero_vmem_rows(buf, n_rows, row_words):
    nv = row_words // _LANES

    def body(i, c):
        for j in range(nv):
            buf[i, pl.ds(j * _LANES, _LANES)] = jnp.zeros((_LANES,), jnp.float32)
        return c

    lax.fori_loop(0, n_rows, body, 0)


def _fill_vmem_rows(buf, n_rows, row_words, value):
    nv = row_words // _LANES

    def body(i, c):
        for j in range(nv):
            buf[i, pl.ds(j * _LANES, _LANES)] = jnp.full((_LANES,), value, jnp.float32)
        return c

    lax.fori_loop(0, n_rows, body, 0)


def _chunk_loop(n_chunks, sid, fn):
    rounds = (n_chunks + _NS - 1) // _NS

    def body(t, c):
        ch = t * _NS + sid

        @pl.when(ch < n_chunks)
        def _():
            fn(ch)

        return c

    lax.fori_loop(0, rounds, body, 0)


_DEG_W = 128


def _edge_blocks(n_batches):
    per_tile = -(-n_batches // _NW) // 8 * 8 + 8
    last_tile = n_batches - per_tile * (_NW - 1)
    assert 0 < last_tile <= per_tile

    def blocks_of(total):
        out = []
        while total >= 24:
            out.append(24)
            total -= 24
        while total > 0:
            out.append(min(8, total))
            total -= min(8, total)
        assert all(b % 2 == 0 for b in out)
        return out

    return per_tile, blocks_of(per_tile), blocks_of(last_tile)


def _make_deg_kernel(n_nodes, n_edges):
    n_batches = n_edges // _EDGE_BATCH
    per_tile, blocks_main, blocks_last = _edge_blocks(n_batches)
    max_blk = max(blocks_main)
    n_chunks = n_nodes // _ROW_CHUNK
    assert n_nodes % _ROW_CHUNK == 0

    mesh = plsc.VectorSubcoreMesh(core_axis_name="c", subcore_axis_name="s")

    @functools.partial(
        pl.kernel,
        mesh=mesh,
        out_type=jax.ShapeDtypeStruct((_NC, n_nodes, _DEG_W), jnp.float32),
        scratch_types=[
            pltpu.VMEM_SHARED((n_nodes, _DEG_W), jnp.float32),
            pltpu.VMEM((_ROW_CHUNK, _DEG_W), jnp.float32),
            pltpu.VMEM((_EDGE_BATCH, _DEG_W), jnp.float32),
            pltpu.VMEM((_EDGE_BATCH, _DEG_W), jnp.float32),
            pltpu.VMEM((max_blk, _EDGE_BATCH), jnp.int32),
            pltpu.VMEM((max_blk, _EDGE_BATCH), jnp.int32),
            pltpu.SemaphoreType.DMA,
        ],
    )
    def deg_kernel(src_hbm, dst_hbm, out_hbm, acc, bounce, ones_s, ones_d,
                   sblk, dblk, sem):
        cid = lax.axis_index("c")
        sid = lax.axis_index("s")
        wid = sid * _NC + cid

        _zero_vmem_rows(bounce, _ROW_CHUNK, _DEG_W)

        def zero_chunk(ch):
            r0 = pl.multiple_of(ch * _ROW_CHUNK, 8)
            pltpu.sync_copy(bounce, acc.at[pl.ds(r0, _ROW_CHUNK)])

        _chunk_loop(n_chunks, sid, zero_chunk)

        half = _DEG_W // 2
        nv = _DEG_W // _LANES

        def fill(i, c):
            for j in range(nv):
                v = 1.0 if j * _LANES < half else 0.0
                ones_s[i, pl.ds(j * _LANES, _LANES)] = jnp.full(
                    (_LANES,), v, jnp.float32)
                ones_d[i, pl.ds(j * _LANES, _LANES)] = jnp.full(
                    (_LANES,), 1.0 - v, jnp.float32)
            return c

        lax.fori_loop(0, _EDGE_BATCH, fill, 0)
        plsc.subcore_barrier()

        def run_block(row0, blk):
            pltpu.sync_copy(src_hbm.at[pl.ds(row0, blk)],
                            sblk.at[pl.ds(0, blk)])
            pltpu.sync_copy(dst_hbm.at[pl.ds(row0, blk)],
                            dblk.at[pl.ds(0, blk)])

            def issue(j, c):
                pltpu.async_copy(ones_s, acc.at[sblk.at[j]], sem, add=True)
                pltpu.async_copy(ones_d, acc.at[dblk.at[j]], sem, add=True)
                return c

            lax.fori_loop(0, blk, issue, 0)

            def drain(j, c):
                pltpu.make_async_copy(ones_s, acc.at[sblk.at[0]], sem).wait()
                pltpu.make_async_copy(ones_s, acc.at[sblk.at[0]], sem).wait()
                return c

            lax.fori_loop(0, blk, drain, 0)

        @pl.when(wid < _NW - 1)
        def _():
            base = pl.multiple_of(wid * per_tile, 8)
            off = 0
            for blk in blocks_main:
                run_block(base + off, blk)
                off += blk

        @pl.when(wid == _NW - 1)
        def _():
            off = per_tile * (_NW - 1)
            for blk in blocks_last:
                run_block(off, blk)
                off += blk

        plsc.subcore_barrier()

        def drain_chunk(ch):
            r0 = pl.multiple_of(ch * _ROW_CHUNK, 8)
            pltpu.sync_copy(acc.at[pl.ds(r0, _ROW_CHUNK)], bounce)
            pltpu.sync_copy(bounce, out_hbm.at[cid, pl.ds(r0, _ROW_CHUNK)])

        _chunk_loop(n_chunks, sid, drain_chunk)

    return deg_kernel


def _make_scatter_kernel(n_nodes, n_edges, d):
    n_batches = n_edges // _EDGE_BATCH
    assert n_edges % _EDGE_BATCH == 0
    per_tile, blocks_main, blocks_last = _edge_blocks(n_batches)
    max_blk = max(blocks_main)

    n_chunks = n_nodes // _ROW_CHUNK
    assert n_nodes % _ROW_CHUNK == 0

    mesh = plsc.VectorSubcoreMesh(core_axis_name="c", subcore_axis_name="s")

    @functools.partial(
        pl.kernel,
        mesh=mesh,
        out_type=jax.ShapeDtypeStruct((_NC, n_nodes, d), jnp.float32),
        scratch_types=[
            pltpu.VMEM_SHARED((n_nodes, d), jnp.float32),
            pltpu.VMEM((_ROW_CHUNK, d), jnp.float32),
            pltpu.VMEM((_EDGE_BATCH, d), jnp.float32),
            pltpu.VMEM((_EDGE_BATCH, d), jnp.float32),
            pltpu.VMEM((max_blk, _EDGE_BATCH), jnp.int32),
            pltpu.VMEM((max_blk, _EDGE_BATCH), jnp.int32),
            pltpu.SemaphoreType.DMA,
            pltpu.SemaphoreType.DMA,
            pltpu.SemaphoreType.DMA,
            pltpu.SemaphoreType.DMA,
        ],
    )
    def scatter_kernel(h_hbm, src_hbm, dst_hbm, out_hbm, acc, bounce,
                       g0, g1, sblk, dblk, gs0, gs1, ss0, ss1):
        cid = lax.axis_index("c")
        sid = lax.axis_index("s")
        wid = sid * _NC + cid

        _zero_vmem_rows(bounce, _ROW_CHUNK, d)

        def zero_chunk(ch):
            r0 = pl.multiple_of(ch * _ROW_CHUNK, 8)
            pltpu.sync_copy(bounce, acc.at[pl.ds(r0, _ROW_CHUNK)])

        _chunk_loop(n_chunks, sid, zero_chunk)
        plsc.subcore_barrier()

        def run_block(row0, blk):
            pltpu.sync_copy(src_hbm.at[pl.ds(row0, blk)],
                            sblk.at[pl.ds(0, blk)])
            pltpu.sync_copy(dst_hbm.at[pl.ds(row0, blk)],
                            dblk.at[pl.ds(0, blk)])
            pltpu.async_copy(h_hbm.at[sblk.at[0]], g0, gs0)

            def pair(t, c):
                j0 = 2 * t
                pltpu.async_copy(h_hbm.at[sblk.at[j0 + 1]], g1, gs1)
                pltpu.make_async_copy(h_hbm.at[sblk.at[j0]], g0, gs0).wait()
                pltpu.async_copy(g0, acc.at[dblk.at[j0]], ss0, add=True)

                @pl.when(t < blk // 2 - 1)
                def _():
                    pltpu.make_async_copy(
                        g0, acc.at[dblk.at[j0]], ss0).wait()
                    pltpu.async_copy(h_hbm.at[sblk.at[j0 + 2]], g0, gs0)

                pltpu.make_async_copy(h_hbm.at[sblk.at[j0 + 1]], g1, gs1).wait()
                pltpu.async_copy(g1, acc.at[dblk.at[j0 + 1]], ss1, add=True)

                @pl.when(t < blk // 2 - 1)
                def _():
                    pltpu.make_async_copy(
                        g1, acc.at[dblk.at[j0 + 1]], ss1).wait()

                return c

            lax.fori_loop(0, blk // 2, pair, 0)
            pltpu.make_async_copy(g0, acc.at[dblk.at[0]], ss0).wait()
            pltpu.make_async_copy(g1, acc.at[dblk.at[0]], ss1).wait()

        @pl.when(wid < _NW - 1)
        def _():
            base = pl.multiple_of(wid * per_tile, 8)
            off = 0
            for blk in blocks_main:
                run_block(base + off, blk)
                off += blk

        @pl.when(wid == _NW - 1)
        def _():
            off = per_tile * (_NW - 1)
            for blk in blocks_last:
                run_block(off, blk)
                off += blk

        plsc.subcore_barrier()

        def drain_chunk(ch):
            r0 = pl.multiple_of(ch * _ROW_CHUNK, 8)
            pltpu.sync_copy(acc.at[pl.ds(r0, _ROW_CHUNK)], bounce)
            pltpu.sync_copy(bounce, out_hbm.at[cid, pl.ds(r0, _ROW_CHUNK)])

        _chunk_loop(n_chunks, sid, drain_chunk)

    return scatter_kernel


_ROW_BLOCK = 2000


def _mm_plain_body(x_ref, w_ref, o_ref):
    o_ref[...] = jnp.dot(x_ref[...], w_ref[...],
                         preferred_element_type=jnp.float32,
                         precision=lax.Precision.HIGHEST)


def _mm_plain(x, w):
    n, din = x.shape
    dout = w.shape[1]
    grid = (n // _ROW_BLOCK,)
    return pl.pallas_call(
        _mm_plain_body,
        grid=grid,
        in_specs=[
            pl.BlockSpec((_ROW_BLOCK, din), lambda i: (i, 0)),
            pl.BlockSpec((din, dout), lambda i: (0, 0)),
        ],
        out_specs=pl.BlockSpec((_ROW_BLOCK, dout), lambda i: (i, 0)),
        out_shape=jax.ShapeDtypeStruct((n, dout), jnp.float32),
    )(x, w)


def _scale_body(a_ref, deg_ref, o_ref):
    onorm = lax.rsqrt(jnp.maximum(deg_ref[...], 1.0))
    o_ref[...] = a_ref[...] * onorm


def _scale_rows(a, out_deg):
    n, d = a.shape
    grid = (n // _ROW_BLOCK,)
    return pl.pallas_call(
        _scale_body,
        grid=grid,
        in_specs=[
            pl.BlockSpec((_ROW_BLOCK, d), lambda i: (i, 0)),
            pl.BlockSpec((_ROW_BLOCK, 1), lambda i: (i, 0)),
        ],
        out_specs=pl.BlockSpec((_ROW_BLOCK, d), lambda i: (i, 0)),
        out_shape=jax.ShapeDtypeStruct((n, d), jnp.float32),
    )(a, out_deg)


def _mm_mid_body(a0_ref, a1_ref, ideg_ref, odeg_ref, b_ref, w_ref, o_ref):
    inorm = lax.rsqrt(jnp.maximum(ideg_ref[...], 1.0))
    onorm = lax.rsqrt(jnp.maximum(odeg_ref[...], 1.0))
    rst = (a0_ref[...] + a1_ref[...]) * inorm + b_ref[...]
    rst = jnp.maximum(rst, 0.0)
    o_ref[...] = jnp.dot(rst * onorm, w_ref[...],
                         preferred_element_type=jnp.float32,
                         precision=lax.Precision.HIGHEST)


def _mm_mid(a0, a1, in_deg, out_deg, b, w):
    n, din = a0.shape
    dout = w.shape[1]
    grid = (n // _ROW_BLOCK,)
    return pl.pallas_call(
        _mm_mid_body,
        grid=grid,
        in_specs=[
            pl.BlockSpec((_ROW_BLOCK, din), lambda i: (i, 0)),
            pl.BlockSpec((_ROW_BLOCK, din), lambda i: (i, 0)),
            pl.BlockSpec((_ROW_BLOCK, 1), lambda i: (i, 0)),
            pl.BlockSpec((_ROW_BLOCK, 1), lambda i: (i, 0)),
            pl.BlockSpec((1, din), lambda i: (0, 0)),
            pl.BlockSpec((din, dout), lambda i: (0, 0)),
        ],
        out_specs=pl.BlockSpec((_ROW_BLOCK, dout), lambda i: (i, 0)),
        out_shape=jax.ShapeDtypeStruct((n, dout), jnp.float32),
    )(a0, a1, in_deg, out_deg, b, w)


def _mm_last_body(dout, a0_ref, a1_ref, ideg_ref, b_ref, o_ref):
    inorm = lax.rsqrt(jnp.maximum(ideg_ref[...], 1.0))
    agg = (a0_ref[...] + a1_ref[...])[:, :dout]
    o_ref[...] = agg * inorm + b_ref[...]


def _mm_last(a0, a1, in_deg, b):
    n, dpad = a0.shape
    dout = b.shape[1]
    grid = (n // _ROW_BLOCK,)
    return pl.pallas_call(
        functools.partial(_mm_last_body, dout),
        grid=grid,
        in_specs=[
            pl.BlockSpec((_ROW_BLOCK, dpad), lambda i: (i, 0)),
            pl.BlockSpec((_ROW_BLOCK, dpad), lambda i: (i, 0)),
            pl.BlockSpec((_ROW_BLOCK, 1), lambda i: (i, 0)),
            pl.BlockSpec((1, dout), lambda i: (0, 0)),
        ],
        out_specs=pl.BlockSpec((_ROW_BLOCK, dout), lambda i: (i, 0)),
        out_shape=jax.ShapeDtypeStruct((n, dout), jnp.float32),
    )(a0, a1, in_deg, b)


def kernel(x, edge_index, W0, b0, W1, b1, W2, b2):
    n, din = x.shape
    e = edge_index.shape[1]
    hid = W0.shape[1]
    ncls = W2.shape[1]

    src2 = edge_index[0].reshape(e // _EDGE_BATCH, _EDGE_BATCH)
    dst2 = edge_index[1].reshape(e // _EDGE_BATCH, _EDGE_BATCH)

    mm0 = _mm_plain(x, W0)

    deg_parts = _make_deg_kernel(n, e)(src2, dst2)
    out_deg = (deg_parts[0, :, 0] + deg_parts[1, :, 0]).reshape(n, 1)
    in_deg = (deg_parts[0, :, _DEG_W // 2]
              + deg_parts[1, :, _DEG_W // 2]).reshape(n, 1)

    scat_hid = _make_scatter_kernel(n, e, hid)

    W2p = jnp.pad(W2, ((0, 0), (0, hid - ncls)))

    h0 = _scale_rows(mm0, out_deg)
    p0 = scat_hid(h0, src2, dst2)
    h1 = _mm_mid(p0[0], p0[1], in_deg, out_deg, b0.reshape(1, -1), W1)
    p1 = scat_hid(h1, src2, dst2)
    h2 = _mm_mid(p1[0], p1[1], in_deg, out_deg, b1.reshape(1, -1), W2p)
    p2 = scat_hid(h2, src2, dst2)
    out = _mm_last(p2[0], p2[1], in_deg, b2.reshape(1, -1))
    return out
